# Initial kernel scaffold; baseline (speedup 1.0000x reference)
import jax
import jax.numpy as jnp
from jax.experimental import pallas as pl


def kernel(token_centers_lookup, query_tokens):
    raise NotImplementedError("write your pallas kernel here")


# speedup vs baseline: 15.8108x; 15.8108x over previous
import functools

import jax
import jax.numpy as jnp
from jax import lax
from jax.experimental import pallas as pl
from jax.experimental.pallas import tpu as pltpu
from jax.experimental.pallas import tpu_sc as plsc

_NUM_CORES = 2
_NUM_SUBCORES = 16
_NW = _NUM_CORES * _NUM_SUBCORES
_B, _T, _C = 16384, 200, 6
_TOKENS = _B * _T
_TPW = _TOKENS // _NW
_CN = 4096
_NCHUNK = _TPW // _CN
_TABLE = 2560


def _make_gather():
    mesh = plsc.VectorSubcoreMesh(core_axis_name="c", subcore_axis_name="s")

    @functools.partial(
        pl.kernel,
        mesh=mesh,
        out_type=jax.ShapeDtypeStruct((_TOKENS * 2,), jnp.float32),
        scratch_types=[
            pltpu.VMEM((_TABLE,), jnp.float32),
            pltpu.VMEM((_CN * _C,), jnp.int32),
            pltpu.VMEM((_CN * 2,), jnp.float32),
        ],
        compiler_params=pltpu.CompilerParams(needs_layout_passes=False),
    )
    def k(table_hbm, q_hbm, out_hbm, table_v, qbuf_v, obuf_v):
        wid = lax.axis_index("s") * _NUM_CORES + lax.axis_index("c")
        pltpu.sync_copy(table_hbm, table_v)

        lane = lax.iota(jnp.int32, 16)
        offvec = 6 * (lane >> 1) + 1 + (lane & 1)

        def chunk_body(c, _):
            tok0 = wid * _TPW + c * _CN
            pltpu.sync_copy(q_hbm.at[pl.ds(tok0 * _C, _CN * _C)], qbuf_v)

            def body(i, _):
                base = i * 16
                qidx = offvec + 3 * base
                tok = plsc.load_gather(qbuf_v, [qidx])
                vals = plsc.load_gather(table_v, [tok])
                obuf_v[pl.ds(base, 16)] = vals
                return ()

            lax.fori_loop(0, _CN * 2 // 16, body, ())
            pltpu.sync_copy(obuf_v, out_hbm.at[pl.ds(tok0 * 2, _CN * 2)])
            return ()

        lax.fori_loop(0, _NCHUNK, chunk_body, ())

    return k


_gather = _make_gather()


def kernel(token_centers_lookup, query_tokens):
    q_flat = query_tokens.reshape(-1)
    out = _gather(token_centers_lookup, q_flat)
    return out.reshape(_B, _T, 2)


# speedup vs baseline: 15.9601x; 1.0094x over previous
import functools

import jax
import jax.numpy as jnp
from jax import lax
from jax.experimental import pallas as pl
from jax.experimental.pallas import tpu as pltpu
from jax.experimental.pallas import tpu_sc as plsc

_NUM_CORES = 2
_NUM_SUBCORES = 16
_NW = _NUM_CORES * _NUM_SUBCORES
_B, _T, _C = 16384, 200, 6
_TOKENS = _B * _T
_TPW = _TOKENS // _NW
_CN = 6400
_NCHUNK = _TPW // _CN
_NPAIR = _NCHUNK // 2
_TABLE = 2560
_UNROLL = 8
_ITERS = _CN * 2 // 16


def _make_gather():
    mesh = plsc.VectorSubcoreMesh(core_axis_name="c", subcore_axis_name="s")

    @functools.partial(
        pl.kernel,
        mesh=mesh,
        out_type=jax.ShapeDtypeStruct((_TOKENS * 2,), jnp.float32),
        scratch_types=[
            pltpu.VMEM((_TABLE,), jnp.float32),
            pltpu.VMEM((_CN * _C,), jnp.int32),
            pltpu.VMEM((_CN * _C,), jnp.int32),
            pltpu.VMEM((_CN * 2,), jnp.float32),
            pltpu.VMEM((_CN * 2,), jnp.float32),
            pltpu.SemaphoreType.DMA,
            pltpu.SemaphoreType.DMA,
            pltpu.SemaphoreType.DMA,
            pltpu.SemaphoreType.DMA,
        ],
        compiler_params=pltpu.CompilerParams(needs_layout_passes=False),
    )
    def k(table_hbm, q_hbm, out_hbm, table_v, qb0, qb1, ob0, ob1, si0, si1, so0, so1):
        wid = lax.axis_index("s") * _NUM_CORES + lax.axis_index("c")
        tok_base = wid * _TPW
        pltpu.sync_copy(table_hbm, table_v)

        qbufs = (qb0, qb1)
        obufs = (ob0, ob1)
        sin = (si0, si1)
        sout = (so0, so1)

        lane = lax.iota(jnp.int32, 16)
        offvec = 6 * (lane >> 1) + 1 + (lane & 1)

        def in_copy(chunk, b):
            tok0 = tok_base + chunk * _CN
            return pltpu.make_async_copy(
                q_hbm.at[pl.ds(tok0 * _C, _CN * _C)], qbufs[b], sin[b]
            )

        def out_copy(chunk, b):
            tok0 = tok_base + chunk * _CN
            return pltpu.make_async_copy(
                obufs[b], out_hbm.at[pl.ds(tok0 * 2, _CN * 2)], sout[b]
            )

        in_copy(0, 0).start()
        in_copy(1, 1).start()

        def pair_body(pair, _):
            for b in range(2):
                chunk = pair * 2 + b
                in_copy(chunk, b).wait()

                @pl.when(pair > 0)
                def _():
                    out_copy(chunk - 2, b).wait()

                qbuf = qbufs[b]
                obuf = obufs[b]

                def body(i, _):
                    for u in range(_UNROLL):
                        base = (i * _UNROLL + u) * 16
                        qidx = offvec + 3 * base
                        tok = plsc.load_gather(qbuf, [qidx])
                        vals = plsc.load_gather(table_v, [tok])
                        obuf[pl.ds(base, 16)] = vals
                    return ()

                lax.fori_loop(0, _ITERS // _UNROLL, body, ())
                out_copy(chunk, b).start()

                @pl.when(pair < _NPAIR - 1)
                def _():
                    in_copy(chunk + 2, b).start()

            return ()

        lax.fori_loop(0, _NPAIR, pair_body, ())
        out_copy(_NCHUNK - 2, 0).wait()
        out_copy(_NCHUNK - 1, 1).wait()

    return k


_gather = _make_gather()


def kernel(token_centers_lookup, query_tokens):
    q_flat = query_tokens.reshape(-1)
    out = _gather(token_centers_lookup, q_flat)
    return out.reshape(_B, _T, 2)
